# baseline (speedup 1.0000x reference)
import functools

import jax
import jax.numpy as jnp
from jax import lax
from jax.experimental import pallas as pl
from jax.experimental.pallas import tpu as pltpu
from jax.experimental.pallas import tpu_sc as plsc

_VOCAB = 100000
_DIM = 128
_MDIM = 2048
_B, _S = 4, 4096
_N = _B * _S
_NW = 32
_PER_W = _N // _NW
_CHUNK = 128
_NCH = _PER_W // _CHUNK
_MOD = _VOCAB - 1


def _sc_hash_gather(tokens_flat, embed_w):
    mesh = plsc.VectorSubcoreMesh(core_axis_name="c", subcore_axis_name="s")

    @functools.partial(
        pl.kernel,
        out_type=jax.ShapeDtypeStruct((_N, _DIM), jnp.float32),
        mesh=mesh,
        scratch_types=[
            pltpu.VMEM((_PER_W + 16,), jnp.int32),
            pltpu.VMEM((_NCH, _CHUNK), jnp.int32),
            pltpu.VMEM((_PER_W, _DIM), jnp.float32),
            pltpu.SemaphoreType.DMA,
            pltpu.SemaphoreType.DMA,
        ],
    )
    def k(tok_hbm, table_hbm, h_hbm, tok_v, idx_v, rows_v, gsem, wsem):
        wid = lax.axis_index("s") * 2 + lax.axis_index("c")
        base = wid * _PER_W
        pltpu.sync_copy(tok_hbm.at[pl.ds(base, _PER_W)],
                        tok_v.at[pl.ds(16, _PER_W)])

        @pl.when(wid != 0)
        def _():
            pltpu.sync_copy(tok_hbm.at[pl.ds(base - 8, 8)],
                            tok_v.at[pl.ds(8, 8)])

        not_start = jnp.minimum((wid * _PER_W) % _S, 1)
        lane = lax.iota(jnp.int32, 16)
        gathers = []
        for j in range(_NCH):
            for v in range(_CHUNK // 16):
                k16 = j * (_CHUNK // 16) + v
                curr = tok_v[pl.ds(16 + k16 * 16, 16)]
                prev = tok_v[pl.ds(15 + k16 * 16, 16)]
                h = (36313 * curr) ^ (27191 * prev)
                if k16 == 0:
                    first_mask = (lane + not_start) == 0
                    h = jnp.where(first_mask, 36313 * curr, h)
                idx_v[j, pl.ds(v * 16, 16)] = h % _MOD
            gathers.append(
                pltpu.async_copy(table_hbm.at[idx_v.at[j]],
                                 rows_v.at[pl.ds(j * _CHUNK, _CHUNK)], gsem))
        writes = []
        for j in range(_NCH):
            gathers[j].wait()
            writes.append(
                pltpu.async_copy(rows_v.at[pl.ds(j * _CHUNK, _CHUNK)],
                                 h_hbm.at[pl.ds(base + j * _CHUNK, _CHUNK)],
                                 wsem))
        for w in writes:
            w.wait()

    return k(tokens_flat, embed_w)


def _tc_project(h, proj_w, scale):
    bm = 2048

    def mm(scale_ref, x_ref, w_ref, o_ref):
        x = x_ref[...].astype(jnp.bfloat16)
        w = w_ref[...].astype(jnp.bfloat16)
        acc = lax.dot_general(x, w, (((1,), (1,)), ((), ())),
                              preferred_element_type=jnp.float32)
        o_ref[...] = acc * scale_ref[0]

    return pl.pallas_call(
        mm,
        grid=(_N // bm,),
        in_specs=[
            pl.BlockSpec(memory_space=pltpu.SMEM),
            pl.BlockSpec((bm, _DIM), lambda i: (i, 0)),
            pl.BlockSpec((_MDIM, _DIM), lambda i: (0, 0)),
        ],
        out_specs=pl.BlockSpec((bm, _MDIM), lambda i: (i, 0)),
        out_shape=jax.ShapeDtypeStruct((_N, _MDIM), jnp.float32),
    )(scale.reshape(1), h, proj_w)


def kernel(token_ids, embed_w, proj_w, scale):
    tokens_flat = token_ids.reshape(_N)
    h = _sc_hash_gather(tokens_flat, embed_w)
    out = _tc_project(h, proj_w, scale)
    return out.reshape(_B, _S, _MDIM)


# speedup vs baseline: 1.0231x; 1.0231x over previous
import functools

import jax
import jax.numpy as jnp
from jax import lax
from jax.experimental import pallas as pl
from jax.experimental.pallas import tpu as pltpu
from jax.experimental.pallas import tpu_sc as plsc

_VOCAB = 100000
_DIM = 128
_MDIM = 2048
_B, _S = 4, 4096
_N = _B * _S
_NW = 32
_PER_W = _N // _NW
_CHUNK = 64
_NCH = _PER_W // _CHUNK
_MOD = _VOCAB - 1


def _sc_hash_gather(tokens_flat, embed_w):
    mesh = plsc.VectorSubcoreMesh(core_axis_name="c", subcore_axis_name="s")

    @functools.partial(
        pl.kernel,
        out_type=jax.ShapeDtypeStruct((_N, _DIM), jnp.float32),
        mesh=mesh,
        scratch_types=[
            pltpu.VMEM((_PER_W + 16,), jnp.int32),
            pltpu.VMEM((_NCH, _CHUNK), jnp.int32),
            pltpu.VMEM((_PER_W, _DIM), jnp.float32),
            pltpu.SemaphoreType.DMA,
            pltpu.SemaphoreType.DMA,
        ],
    )
    def k(tok_hbm, table_hbm, h_hbm, tok_v, idx_v, rows_v, gsem, wsem):
        wid = lax.axis_index("s") * 2 + lax.axis_index("c")
        base = wid * _PER_W
        pltpu.sync_copy(tok_hbm.at[pl.ds(base, _PER_W)],
                        tok_v.at[pl.ds(16, _PER_W)])

        @pl.when(wid != 0)
        def _():
            pltpu.sync_copy(tok_hbm.at[pl.ds(base - 8, 8)],
                            tok_v.at[pl.ds(8, 8)])

        not_start = jnp.minimum((wid * _PER_W) % _S, 1)
        lane = lax.iota(jnp.int32, 16)
        gathers = []
        for j in range(_NCH):
            for v in range(_CHUNK // 16):
                k16 = j * (_CHUNK // 16) + v
                curr = tok_v[pl.ds(16 + k16 * 16, 16)]
                prev = tok_v[pl.ds(15 + k16 * 16, 16)]
                h = (36313 * curr) ^ (27191 * prev)
                if k16 == 0:
                    first_mask = (lane + not_start) == 0
                    h = jnp.where(first_mask, 36313 * curr, h)
                idx_v[j, pl.ds(v * 16, 16)] = h % _MOD
            gathers.append(
                pltpu.async_copy(table_hbm.at[idx_v.at[j]],
                                 rows_v.at[pl.ds(j * _CHUNK, _CHUNK)], gsem))
        writes = []
        for j in range(_NCH):
            gathers[j].wait()
            writes.append(
                pltpu.async_copy(rows_v.at[pl.ds(j * _CHUNK, _CHUNK)],
                                 h_hbm.at[pl.ds(base + j * _CHUNK, _CHUNK)],
                                 wsem))
        for w in writes:
            w.wait()

    return k(tokens_flat, embed_w)


def _tc_project(h, proj_w, scale):
    bm = 1024

    def mm(scale_ref, x_ref, w_ref, o_ref):
        x = x_ref[...].astype(jnp.bfloat16)
        w = w_ref[...].astype(jnp.bfloat16)
        acc = lax.dot_general(x, w, (((1,), (1,)), ((), ())),
                              preferred_element_type=jnp.float32)
        o_ref[...] = acc * scale_ref[0]

    return pl.pallas_call(
        mm,
        grid=(_N // bm,),
        in_specs=[
            pl.BlockSpec(memory_space=pltpu.SMEM),
            pl.BlockSpec((bm, _DIM), lambda i: (i, 0)),
            pl.BlockSpec((_MDIM, _DIM), lambda i: (0, 0)),
        ],
        out_specs=pl.BlockSpec((bm, _MDIM), lambda i: (i, 0)),
        out_shape=jax.ShapeDtypeStruct((_N, _MDIM), jnp.float32),
    )(scale.reshape(1), h, proj_w)


def kernel(token_ids, embed_w, proj_w, scale):
    tokens_flat = token_ids.reshape(_N)
    h = _sc_hash_gather(tokens_flat, embed_w)
    out = _tc_project(h, proj_w, scale)
    return out.reshape(_B, _S, _MDIM)


# speedup vs baseline: 1.0239x; 1.0008x over previous
import functools

import jax
import jax.numpy as jnp
from jax import lax
from jax.experimental import pallas as pl
from jax.experimental.pallas import tpu as pltpu
from jax.experimental.pallas import tpu_sc as plsc

_VOCAB = 100000
_DIM = 128
_MDIM = 2048
_B, _S = 4, 4096
_N = _B * _S
_NW = 32
_PER_W = _N // _NW
_CHUNK = 32
_NCH = _PER_W // _CHUNK
_MOD = _VOCAB - 1


def _sc_hash_gather(tokens_flat, embed_w):
    mesh = plsc.VectorSubcoreMesh(core_axis_name="c", subcore_axis_name="s")

    @functools.partial(
        pl.kernel,
        out_type=jax.ShapeDtypeStruct((_N, _DIM), jnp.float32),
        mesh=mesh,
        scratch_types=[
            pltpu.VMEM((_PER_W + 16,), jnp.int32),
            pltpu.VMEM((_NCH, _CHUNK), jnp.int32),
            pltpu.VMEM((_PER_W, _DIM), jnp.float32),
            pltpu.SemaphoreType.DMA,
            pltpu.SemaphoreType.DMA,
        ],
    )
    def k(tok_hbm, table_hbm, h_hbm, tok_v, idx_v, rows_v, gsem, wsem):
        wid = lax.axis_index("s") * 2 + lax.axis_index("c")
        base = wid * _PER_W
        pltpu.sync_copy(tok_hbm.at[pl.ds(base, _PER_W)],
                        tok_v.at[pl.ds(16, _PER_W)])

        @pl.when(wid != 0)
        def _():
            pltpu.sync_copy(tok_hbm.at[pl.ds(base - 8, 8)],
                            tok_v.at[pl.ds(8, 8)])

        not_start = jnp.minimum((wid * _PER_W) % _S, 1)
        lane = lax.iota(jnp.int32, 16)
        gathers = []
        for j in range(_NCH):
            for v in range(_CHUNK // 16):
                k16 = j * (_CHUNK // 16) + v
                curr = tok_v[pl.ds(16 + k16 * 16, 16)]
                prev = tok_v[pl.ds(15 + k16 * 16, 16)]
                h = (36313 * curr) ^ (27191 * prev)
                if k16 == 0:
                    first_mask = (lane + not_start) == 0
                    h = jnp.where(first_mask, 36313 * curr, h)
                idx_v[j, pl.ds(v * 16, 16)] = h % _MOD
            gathers.append(
                pltpu.async_copy(table_hbm.at[idx_v.at[j]],
                                 rows_v.at[pl.ds(j * _CHUNK, _CHUNK)], gsem))
        writes = []
        for j in range(_NCH):
            gathers[j].wait()
            writes.append(
                pltpu.async_copy(rows_v.at[pl.ds(j * _CHUNK, _CHUNK)],
                                 h_hbm.at[pl.ds(base + j * _CHUNK, _CHUNK)],
                                 wsem))
        for w in writes:
            w.wait()

    return k(tokens_flat, embed_w)


def _tc_project(h, proj_w, scale):
    bm = 1024

    def mm(scale_ref, x_ref, w_ref, o_ref):
        x = x_ref[...].astype(jnp.bfloat16)
        w = w_ref[...].astype(jnp.bfloat16)
        acc = lax.dot_general(x, w, (((1,), (1,)), ((), ())),
                              preferred_element_type=jnp.float32)
        o_ref[...] = acc * scale_ref[0]

    return pl.pallas_call(
        mm,
        grid=(_N // bm,),
        in_specs=[
            pl.BlockSpec(memory_space=pltpu.SMEM),
            pl.BlockSpec((bm, _DIM), lambda i: (i, 0)),
            pl.BlockSpec((_MDIM, _DIM), lambda i: (0, 0)),
        ],
        out_specs=pl.BlockSpec((bm, _MDIM), lambda i: (i, 0)),
        out_shape=jax.ShapeDtypeStruct((_N, _MDIM), jnp.float32),
    )(scale.reshape(1), h, proj_w)


def kernel(token_ids, embed_w, proj_w, scale):
    tokens_flat = token_ids.reshape(_N)
    h = _sc_hash_gather(tokens_flat, embed_w)
    out = _tc_project(h, proj_w, scale)
    return out.reshape(_B, _S, _MDIM)
